# baseline (speedup 1.0000x reference)
import functools

import jax
import jax.numpy as jnp
from jax import lax
from jax.experimental import pallas as pl
from jax.experimental.pallas import tpu as pltpu
from jax.experimental.pallas import tpu_sc as plsc

B = 4096
EMB = 16
NCAT = 1000
VIS = 512
TM = 512


def _tc_head(scal, visual, uf, itf, cf, Wv, bv2, Wv1):
  def body(scal_ref, visual_ref, uf_ref, itf_ref, cf_ref, Wv_ref, bv_ref,
           Wv1_ref, vemb_ref, fo_ref):
    vis = visual_ref[...]
    vemb_ref[...] = lax.dot_general(
        Wv_ref[...], vis, (((1,), (1,)), ((), ())),
        precision=lax.Precision.HIGHEST,
        preferred_element_type=jnp.float32) + bv_ref[...]
    vlin = lax.dot_general(
        vis, Wv1_ref[...], (((1,), (1,)), ((), ())),
        precision=lax.Precision.HIGHEST,
        preferred_element_type=jnp.float32)
    s0 = scal_ref[1] + scal_ref[3] + scal_ref[5] + scal_ref[6] + scal_ref[7]
    fo_ref[...] = (s0 + scal_ref[0] * uf_ref[...] +
                   scal_ref[2] * itf_ref[...] + scal_ref[4] * cf_ref[...] +
                   vlin)

  return pl.pallas_call(
      body,
      in_specs=[
          pl.BlockSpec(memory_space=pltpu.SMEM),
          pl.BlockSpec((B, VIS), lambda: (0, 0)),
          pl.BlockSpec((B, 1), lambda: (0, 0)),
          pl.BlockSpec((B, 1), lambda: (0, 0)),
          pl.BlockSpec((B, 1), lambda: (0, 0)),
          pl.BlockSpec((EMB, VIS), lambda: (0, 0)),
          pl.BlockSpec((EMB, 1), lambda: (0, 0)),
          pl.BlockSpec((1, VIS), lambda: (0, 0)),
      ],
      out_specs=[
          pl.BlockSpec((EMB, B), lambda: (0, 0)),
          pl.BlockSpec((B, 1), lambda: (0, 0)),
      ],
      out_shape=[
          jax.ShapeDtypeStruct((EMB, B), jnp.float32),
          jax.ShapeDtypeStruct((B, 1), jnp.float32),
      ],
  )(scal, visual, uf, itf, cf, Wv, bv2, Wv1)


def _sc_pairwise(category, ct_t, u_rows_t, i_rows_t, vemb_t):
  info = plsc.get_sparse_core_info()
  nc, ns = info.num_cores, info.num_subcores
  nw = nc * ns
  bpw = B // nw

  mesh = plsc.VectorSubcoreMesh(core_axis_name="c", subcore_axis_name="s")

  @functools.partial(
      pl.kernel,
      mesh=mesh,
      out_type=jax.ShapeDtypeStruct((1, B), jnp.float32),
      scratch_types=[
          pltpu.VMEM((bpw,), jnp.int32),
          pltpu.VMEM((EMB, NCAT), jnp.float32),
          pltpu.VMEM((EMB, bpw), jnp.float32),
          pltpu.VMEM((EMB, bpw), jnp.float32),
          pltpu.VMEM((EMB, bpw), jnp.float32),
          pltpu.VMEM((1, bpw), jnp.float32),
          pltpu.SemaphoreType.DMA,
      ],
      compiler_params=pltpu.CompilerParams(needs_layout_passes=False),
  )
  def pairwise_kernel(cat_hbm, ct_hbm, u_hbm, i_hbm, v_hbm, out_hbm, cidx,
                      ctab, ub, ib, vb, pwb, sem):
    wid = lax.axis_index("s") * nc + lax.axis_index("c")
    base = wid * bpw
    cps = [
        pltpu.async_copy(cat_hbm.at[pl.ds(base, bpw)], cidx, sem),
        pltpu.async_copy(ct_hbm, ctab, sem),
        pltpu.async_copy(u_hbm.at[:, pl.ds(base, bpw)], ub, sem),
        pltpu.async_copy(i_hbm.at[:, pl.ds(base, bpw)], ib, sem),
        pltpu.async_copy(v_hbm.at[:, pl.ds(base, bpw)], vb, sem),
    ]
    for cp in cps:
      cp.wait()

    for g in range(bpw // 16):
      sl = pl.ds(g * 16, 16)
      cvec = cidx[sl]
      acc = jnp.zeros((16,), jnp.float32)
      for e in range(EMB):
        ev = jnp.full((16,), e, jnp.int32)
        cv = plsc.load_gather(ctab, [ev, cvec])
        acc = acc + ub[e, sl] * ib[e, sl] * cv * vb[e, sl]
      pwb[0, sl] = acc
    pltpu.sync_copy(pwb, out_hbm.at[:, pl.ds(base, bpw)])

  return pairwise_kernel(category, ct_t, u_rows_t, i_rows_t, vemb_t)


def _tc_broadcast(fo, pw):
  nb = B // TM

  def body(fo_ref, pw_ref, out_ref):
    out_ref[...] = fo_ref[...] + pw_ref[...]

  return pl.pallas_call(
      body,
      grid=(nb,),
      in_specs=[
          pl.BlockSpec((TM, 1), lambda k: (k, 0)),
          pl.BlockSpec((1, B), lambda k: (0, 0)),
      ],
      out_specs=pl.BlockSpec((TM, B), lambda k: (k, 0)),
      out_shape=jax.ShapeDtypeStruct((B, B), jnp.float32),
      compiler_params=pltpu.CompilerParams(
          dimension_semantics=("arbitrary",)),
  )(fo, pw)


def kernel(user, item, category, visual, user_table, item_table, cat_table,
           Wv, bv, Wu, bu, Wi, bi, Wc, bc, Wv1, bv1, bias):
  u_rows_t = jnp.take(user_table, user, axis=0).T
  i_rows_t = jnp.take(item_table, item, axis=0).T
  scal = jnp.concatenate([
      Wu.reshape(-1), bu.reshape(-1), Wi.reshape(-1), bi.reshape(-1),
      Wc.reshape(-1), bc.reshape(-1), bias.reshape(-1), bv1.reshape(-1)
  ])
  uf = user.astype(jnp.float32).reshape(B, 1)
  itf = item.astype(jnp.float32).reshape(B, 1)
  cf = category.astype(jnp.float32).reshape(B, 1)
  vemb_t, fo = _tc_head(scal, visual, uf, itf, cf, Wv, bv.reshape(EMB, 1),
                        Wv1)
  pw = _sc_pairwise(category, cat_table.T, u_rows_t, i_rows_t, vemb_t)
  return _tc_broadcast(fo, pw)


# speedup vs baseline: 1.0089x; 1.0089x over previous
import functools

import jax
import jax.numpy as jnp
from jax import lax
from jax.experimental import pallas as pl
from jax.experimental.pallas import tpu as pltpu
from jax.experimental.pallas import tpu_sc as plsc

B = 4096
EMB = 16
NCAT = 1000
VIS = 512
TM = 512


def _tc_head(scal, visual, Wv, bv2, Wv1):
  def body(scal_ref, visual_ref, Wv_ref, bv_ref, Wv1_ref, vemb_ref, vl_ref):
    vis = visual_ref[...]
    vemb_ref[...] = lax.dot_general(
        Wv_ref[...], vis, (((1,), (1,)), ((), ())),
        precision=lax.Precision.HIGHEST,
        preferred_element_type=jnp.float32) + bv_ref[...]
    s0 = scal_ref[1] + scal_ref[3] + scal_ref[5] + scal_ref[6] + scal_ref[7]
    vl_ref[...] = lax.dot_general(
        Wv1_ref[...], vis, (((1,), (1,)), ((), ())),
        precision=lax.Precision.HIGHEST,
        preferred_element_type=jnp.float32) + s0

  return pl.pallas_call(
      body,
      in_specs=[
          pl.BlockSpec(memory_space=pltpu.SMEM),
          pl.BlockSpec((B, VIS), lambda: (0, 0)),
          pl.BlockSpec((EMB, VIS), lambda: (0, 0)),
          pl.BlockSpec((EMB, 1), lambda: (0, 0)),
          pl.BlockSpec((1, VIS), lambda: (0, 0)),
      ],
      out_specs=[
          pl.BlockSpec((EMB, B), lambda: (0, 0)),
          pl.BlockSpec((1, B), lambda: (0, 0)),
      ],
      out_shape=[
          jax.ShapeDtypeStruct((EMB, B), jnp.float32),
          jax.ShapeDtypeStruct((1, B), jnp.float32),
      ],
  )(scal, visual, Wv, bv2, Wv1)


def _sc_pairwise_fo(user, item, category, scal16, ct_t, u_rows_t, i_rows_t,
                    vemb_t, vl):
  info = plsc.get_sparse_core_info()
  nc, ns = info.num_cores, info.num_subcores
  nw = nc * ns
  bpw = B // nw

  mesh = plsc.VectorSubcoreMesh(core_axis_name="c", subcore_axis_name="s")

  @functools.partial(
      pl.kernel,
      mesh=mesh,
      out_type=[
          jax.ShapeDtypeStruct((1, B), jnp.float32),
          jax.ShapeDtypeStruct((1, B), jnp.float32),
      ],
      scratch_types=[
          pltpu.VMEM((bpw,), jnp.int32),
          pltpu.VMEM((bpw,), jnp.int32),
          pltpu.VMEM((bpw,), jnp.int32),
          pltpu.VMEM((16,), jnp.float32),
          pltpu.VMEM((EMB, NCAT), jnp.float32),
          pltpu.VMEM((EMB, bpw), jnp.float32),
          pltpu.VMEM((EMB, bpw), jnp.float32),
          pltpu.VMEM((EMB, bpw), jnp.float32),
          pltpu.VMEM((1, bpw), jnp.float32),
          pltpu.VMEM((1, bpw), jnp.float32),
          pltpu.VMEM((1, bpw), jnp.float32),
          pltpu.SemaphoreType.DMA,
      ],
      compiler_params=pltpu.CompilerParams(needs_layout_passes=False),
  )
  def pairwise_kernel(user_hbm, item_hbm, cat_hbm, scal_hbm, ct_hbm, u_hbm,
                      i_hbm, v_hbm, vl_hbm, pw_hbm, fo_hbm, uidx, iidx, cidx,
                      scv, ctab, ub, ib, vb, vlb, pwb, fob, sem):
    wid = lax.axis_index("s") * nc + lax.axis_index("c")
    base = wid * bpw
    cps = [
        pltpu.async_copy(user_hbm.at[pl.ds(base, bpw)], uidx, sem),
        pltpu.async_copy(item_hbm.at[pl.ds(base, bpw)], iidx, sem),
        pltpu.async_copy(cat_hbm.at[pl.ds(base, bpw)], cidx, sem),
        pltpu.async_copy(scal_hbm, scv, sem),
        pltpu.async_copy(ct_hbm, ctab, sem),
        pltpu.async_copy(u_hbm.at[:, pl.ds(base, bpw)], ub, sem),
        pltpu.async_copy(i_hbm.at[:, pl.ds(base, bpw)], ib, sem),
        pltpu.async_copy(v_hbm.at[:, pl.ds(base, bpw)], vb, sem),
        pltpu.async_copy(vl_hbm.at[:, pl.ds(base, bpw)], vlb, sem),
    ]
    for cp in cps:
      cp.wait()

    svec = scv[...]
    wu = svec[0]
    wi = svec[2]
    wc = svec[4]
    for g in range(bpw // 16):
      sl = pl.ds(g * 16, 16)
      cvec = cidx[sl]
      acc = jnp.zeros((16,), jnp.float32)
      for e in range(EMB):
        ev = jnp.full((16,), e, jnp.int32)
        cv = plsc.load_gather(ctab, [ev, cvec])
        acc = acc + ub[e, sl] * ib[e, sl] * cv * vb[e, sl]
      pwb[0, sl] = acc
      fob[0, sl] = (vlb[0, sl] + wu * uidx[sl].astype(jnp.float32) +
                    wi * iidx[sl].astype(jnp.float32) +
                    wc * cvec.astype(jnp.float32))
    pltpu.sync_copy(pwb, pw_hbm.at[:, pl.ds(base, bpw)])
    pltpu.sync_copy(fob, fo_hbm.at[:, pl.ds(base, bpw)])

  return pairwise_kernel(user, item, category, scal16, ct_t, u_rows_t,
                         i_rows_t, vemb_t, vl)


def _tc_broadcast(fo, pw):
  nb = B // TM

  def body(fo_ref, pw_ref, out_ref):
    k = pl.program_id(0)
    col = jnp.transpose(fo_ref[0:1, pl.ds(k * TM, TM)], (1, 0))
    out_ref[...] = col + pw_ref[...]

  return pl.pallas_call(
      body,
      grid=(nb,),
      in_specs=[
          pl.BlockSpec((1, B), lambda k: (0, 0)),
          pl.BlockSpec((1, B), lambda k: (0, 0)),
      ],
      out_specs=pl.BlockSpec((TM, B), lambda k: (k, 0)),
      out_shape=jax.ShapeDtypeStruct((B, B), jnp.float32),
      compiler_params=pltpu.CompilerParams(
          dimension_semantics=("arbitrary",)),
  )(fo, pw)


def kernel(user, item, category, visual, user_table, item_table, cat_table,
           Wv, bv, Wu, bu, Wi, bi, Wc, bc, Wv1, bv1, bias):
  u_rows_t = jnp.take(user_table, user, axis=0).T
  i_rows_t = jnp.take(item_table, item, axis=0).T
  zeros8 = jnp.zeros((8,), jnp.float32)
  scal = jnp.concatenate([
      Wu.reshape(-1), bu.reshape(-1), Wi.reshape(-1), bi.reshape(-1),
      Wc.reshape(-1), bc.reshape(-1), bias.reshape(-1), bv1.reshape(-1)
  ])
  scal16 = jnp.concatenate([scal, zeros8])
  vemb_t, vl = _tc_head(scal, visual, Wv, bv.reshape(EMB, 1), Wv1)
  pw, fo = _sc_pairwise_fo(user, item, category, scal16, cat_table.T,
                           u_rows_t, i_rows_t, vemb_t, vl)
  return _tc_broadcast(fo, pw)
